# Initial kernel scaffold; baseline (speedup 1.0000x reference)
import jax
import jax.numpy as jnp
from jax.experimental import pallas as pl


def kernel(x, emb_table, W, b):
    raise NotImplementedError("write your pallas kernel here")


# speedup vs baseline: 8.5683x; 8.5683x over previous
import functools

import jax
import jax.numpy as jnp
from jax import lax
from jax.experimental import pallas as pl
from jax.experimental.pallas import tpu as pltpu
from jax.experimental.pallas import tpu_sc as plsc

_IB = 128


def _fc_body(emb_ref, w_ref, b_ref, out_ref):
    out_ref[...] = (
        jnp.dot(emb_ref[...], w_ref[...], preferred_element_type=jnp.float32)
        + b_ref[...]
    )


def _project_table(emb_table, W, b):
    V, _ = emb_table.shape
    C = W.shape[1]
    return pl.pallas_call(
        _fc_body,
        out_shape=jax.ShapeDtypeStruct((V, C), jnp.float32),
    )(emb_table, W, b.reshape(1, C))


@functools.lru_cache(maxsize=None)
def _make_gather(V, C, N, K):
    info = plsc.get_sparse_core_info()
    NC, NS = info.num_cores, info.num_subcores
    NW = NC * NS
    nblk = N // _IB
    blk_per_w = nblk // NW
    assert nblk * _IB == N and blk_per_w * NW == nblk and blk_per_w % K == 0
    nsteps = blk_per_w // K
    mesh = plsc.VectorSubcoreMesh(core_axis_name="c", subcore_axis_name="s")

    @functools.partial(
        pl.kernel,
        mesh=mesh,
        out_type=jax.ShapeDtypeStruct((nblk, _IB, C), jnp.float32),
        compiler_params=pltpu.CompilerParams(use_tc_tiling_on_sc=False),
        scratch_types=[
            pltpu.VMEM((K, _IB), jnp.int32),
            pltpu.VMEM((K, _IB, C), jnp.float32),
            pltpu.SemaphoreType.DMA,
            pltpu.SemaphoreType.DMA,
        ],
    )
    def gather_kernel(table_hbm, idx_hbm, out_hbm, idx_v, rows_v, isem, gsem):
        wid = lax.axis_index("s") * NC + lax.axis_index("c")
        base = wid * blk_per_w

        def step(i, carry):
            off = base + i * K
            pltpu.async_copy(idx_hbm.at[pl.ds(off, K)], idx_v, isem).wait()
            copies = []
            for j in range(K):
                copies.append(
                    pltpu.async_copy(
                        table_hbm.at[idx_v.at[j]], rows_v.at[j], gsem
                    )
                )
            for c in copies:
                c.wait()
            pltpu.async_copy(rows_v, out_hbm.at[pl.ds(off, K)], isem).wait()
            return carry

        lax.fori_loop(0, nsteps, step, 0, unroll=False)

    return gather_kernel


def kernel(x, emb_table, W, b):
    B, L = x.shape
    V, C = emb_table.shape[0], W.shape[1]
    N = B * L
    Cp = 16
    Wp = jnp.pad(W, ((0, 0), (0, Cp - C)))
    bp = jnp.pad(b, (0, Cp - C))
    table = _project_table(emb_table, Wp, bp)
    idx2d = x.reshape(N // _IB, _IB).astype(jnp.int32)
    out = _make_gather(V, Cp, N, 16)(table, idx2d)
    return out.reshape(B, L, Cp)[:, :, :C]
